# baseline (speedup 1.0000x reference)
import jax
import jax.numpy as jnp
from jax import lax
from jax.experimental import pallas as pl
from jax.experimental.pallas import tpu as pltpu
from jax.experimental.pallas import tpu_sc as plsc

BATCH = 16384
EMBED_DIM = 32
NUM_CORES = 2
NUM_SUBCORES = 16
NUM_WORKERS = NUM_CORES * NUM_SUBCORES
B_PER_W = BATCH // NUM_WORKERS
LANES = 16
ROUNDS = B_PER_W // LANES
SLAB = 8
PASSES = EMBED_DIM // SLAB
STEPS = ROUNDS * PASSES


def _sc_body(u_ids_hbm, i_ids_hbm, ue_t_hbm, ie_t_hbm, ub_hbm, ib_hbm,
             bias_hbm, out_hbm,
             uidx_v, iidx_v, ubufs_v, ibufs_v, ub_v, ib_v,
             bias_v, acc_v, sem0, sem1):
    wid = lax.axis_index("s") * NUM_CORES + lax.axis_index("c")
    base = wid * B_PER_W

    pltpu.sync_copy(u_ids_hbm.at[pl.ds(base, B_PER_W)], uidx_v)
    pltpu.sync_copy(i_ids_hbm.at[pl.ds(base, B_PER_W)], iidx_v)
    pltpu.sync_copy(bias_hbm, bias_v)

    cb = pltpu.make_async_copy(ub_hbm.at[uidx_v], ub_v, sem0)
    ci = pltpu.make_async_copy(ib_hbm.at[iidx_v], ib_v, sem1)
    cb.start()
    ci.start()
    cb.wait()
    ci.wait()

    bias_vec = bias_v[...]
    svec = lax.iota(jnp.int32, LANES)
    sems = (sem0, sem1)

    def fire(step, buf_id):
        r = step // PASSES
        p = step % PASSES
        off = r * LANES
        uvec = uidx_v[pl.ds(off, LANES)]
        ivec = iidx_v[pl.ds(off, LANES)]
        ublk = jnp.bitwise_and(uvec, -128)
        iblk = jnp.bitwise_and(ivec, -128)
        drow = p * SLAB
        sem = sems[0] if buf_id == 0 else sems[1]
        for s in range(LANES):
            uo = pl.multiple_of(ublk[s], 128)
            io = pl.multiple_of(iblk[s], 128)
            pltpu.make_async_copy(
                ue_t_hbm.at[pl.ds(drow, SLAB), pl.ds(uo, 128)],
                ubufs_v.at[buf_id, s], sem).start()
            pltpu.make_async_copy(
                ie_t_hbm.at[pl.ds(drow, SLAB), pl.ds(io, 128)],
                ibufs_v.at[buf_id, s], sem).start()

    def drain(buf_id):
        sem = sems[0] if buf_id == 0 else sems[1]
        def w(s, c2):
            pltpu.make_async_copy(
                ue_t_hbm.at[pl.ds(0, SLAB), pl.ds(0, 128)],
                ubufs_v.at[buf_id, s], sem).wait()
            pltpu.make_async_copy(
                ie_t_hbm.at[pl.ds(0, SLAB), pl.ds(0, 128)],
                ibufs_v.at[buf_id, s], sem).wait()
            return c2
        lax.fori_loop(0, LANES, w, 0)

    def compute(step, buf_id):
        r = step // PASSES
        p = step % PASSES
        off = r * LANES
        ulan = jnp.bitwise_and(uidx_v[pl.ds(off, LANES)], 127)
        ilan = jnp.bitwise_and(iidx_v[pl.ds(off, LANES)], 127)
        acc = acc_v[pl.ds(off, LANES)]
        bvec = jnp.full((LANES,), buf_id, jnp.int32)
        for d in range(SLAB):
            dv = jnp.full((LANES,), d, jnp.int32)
            gu = plsc.load_gather(ubufs_v, [bvec, svec, dv, ulan])
            gi = plsc.load_gather(ibufs_v, [bvec, svec, dv, ilan])
            acc = acc + gu * gi
        acc_v[pl.ds(off, LANES)] = acc

    def init(r, carry):
        off = r * LANES
        acc_v[pl.ds(off, LANES)] = (bias_vec + ub_v[pl.ds(off, LANES)]
                                    + ib_v[pl.ds(off, LANES)])
        return carry
    lax.fori_loop(0, ROUNDS, init, 0)

    fire(0, 0)

    def step_body(k, carry):
        cur = lax.rem(k, 2)

        @pl.when(k + 1 < STEPS)
        def _():
            @pl.when(cur == 0)
            def _():
                fire(k + 1, 1)

            @pl.when(cur == 1)
            def _():
                fire(k + 1, 0)

        @pl.when(cur == 0)
        def _():
            drain(0)
            compute(k, 0)

        @pl.when(cur == 1)
        def _():
            drain(1)
            compute(k, 1)

        return carry

    lax.fori_loop(0, STEPS, step_body, 0)

    pltpu.sync_copy(acc_v, out_hbm.at[pl.ds(base, B_PER_W)])


@jax.jit
def kernel(u_ids, i_ids, user_embeddings, item_embeddings, user_bias,
           item_bias, bias):
    bias16 = jnp.broadcast_to(bias.astype(jnp.float32), (LANES,))
    mesh = plsc.VectorSubcoreMesh(core_axis_name="c", subcore_axis_name="s",
                                  num_cores=NUM_CORES)
    f = pl.kernel(
        _sc_body,
        out_type=jax.ShapeDtypeStruct((BATCH,), jnp.float32),
        mesh=mesh,
        compiler_params=pltpu.CompilerParams(needs_layout_passes=False),
        scratch_types=[
            pltpu.VMEM((B_PER_W,), jnp.int32),
            pltpu.VMEM((B_PER_W,), jnp.int32),
            pltpu.VMEM((2, LANES, SLAB, 128), jnp.float32),
            pltpu.VMEM((2, LANES, SLAB, 128), jnp.float32),
            pltpu.VMEM((B_PER_W,), jnp.float32),
            pltpu.VMEM((B_PER_W,), jnp.float32),
            pltpu.VMEM((LANES,), jnp.float32),
            pltpu.VMEM((B_PER_W,), jnp.float32),
            pltpu.SemaphoreType.DMA,
            pltpu.SemaphoreType.DMA,
        ],
    )
    return f(u_ids, i_ids, user_embeddings.T, item_embeddings.T, user_bias,
             item_bias, bias16)
